# baseline (speedup 1.0000x reference)
import functools

import jax
import jax.numpy as jnp
from jax import lax
from jax.experimental import pallas as pl
from jax.experimental.pallas import tpu as pltpu
from jax.experimental.pallas import tpu_sc as plsc

MARGIN_ = 0.02
N_ = 4096
NW_ = 32
ROWS_ = N_ // NW_
L_ = 16


def _sc_body(p_hbm, y_hbm, out_hbm, pv, yv, numv, denv):
    wid = lax.axis_index("s") * 2 + lax.axis_index("c")
    row0 = wid * ROWS_
    pltpu.sync_copy(p_hbm, pv)
    pltpu.sync_copy(y_hbm, yv)

    def group_body(g, carry):
        acc_n, acc_d = carry
        pr = pv[pl.ds(row0 + g * L_, L_)]
        yr = yv[pl.ds(row0 + g * L_, L_)]
        for l in range(L_):
            p_i = pr[l]
            y_i = yr[l]
            mp_i = MARGIN_ - p_i

            def col_body(c, carry2):
                a_n, a_d = carry2
                pc = pv[pl.ds(c * L_, L_)]
                yc = yv[pl.ds(c * L_, L_)]
                w = jnp.maximum(y_i - yc, 0.0)
                h = jnp.maximum(mp_i + pc, 0.0)
                return a_n + h * w, a_d + w

            acc_n, acc_d = lax.fori_loop(
                0, N_ // L_, col_body, (acc_n, acc_d))
        return acc_n, acc_d

    zero = jnp.zeros((L_,), jnp.float32)
    acc_n, acc_d = lax.fori_loop(0, ROWS_ // L_, group_body, (zero, zero))

    numv[...] = acc_n
    denv[...] = acc_d
    pltpu.sync_copy(numv, out_hbm.at[2 * wid])
    pltpu.sync_copy(denv, out_hbm.at[2 * wid + 1])


@jax.jit
def kernel(pred, y_true):
    p = pred.reshape(-1).astype(jnp.float32)
    y = y_true.reshape(-1).astype(jnp.float32)

    mesh = plsc.VectorSubcoreMesh(core_axis_name="c", subcore_axis_name="s")
    run = functools.partial(
        pl.kernel,
        mesh=mesh,
        out_type=jax.ShapeDtypeStruct((2 * NW_, L_), jnp.float32),
        scratch_types=[
            pltpu.VMEM((N_,), jnp.float32),
            pltpu.VMEM((N_,), jnp.float32),
            pltpu.VMEM((L_,), jnp.float32),
            pltpu.VMEM((L_,), jnp.float32),
        ],
    )(_sc_body)

    parts = run(p, y)
    num = jnp.sum(parts[0::2, :])
    den = jnp.sum(parts[1::2, :])
    return num / (den + 1e-8)


# speedup vs baseline: 5.4704x; 5.4704x over previous
import functools

import jax
import jax.numpy as jnp
from jax import lax
from jax.experimental import pallas as pl
from jax.experimental.pallas import tpu as pltpu

MARGIN_ = 0.02
BT_ = 512
SR_ = 32
SC_ = 128


def _cmpex(x, v, s, k):
    lane = lax.broadcasted_iota(jnp.int32, (SR_, SC_), 1)
    row = lax.broadcasted_iota(jnp.int32, (SR_, SC_), 0)
    if s < SR_:
        d = s
        bs = row & d

        def _xor_rows(a):
            parts = []
            for g in range(0, SR_, 2 * d):
                parts.append(a[g + d:g + 2 * d, :])
                parts.append(a[g:g + d, :])
            return jnp.concatenate(parts, axis=0)

        xp = _xor_rows(x)
        vp = _xor_rows(v)
    else:
        d = s // SR_
        bs = lane & d
        xp = jnp.where(bs == 0, pltpu.roll(x, SC_ - d, 1), pltpu.roll(x, d, 1))
        vp = jnp.where(bs == 0, pltpu.roll(v, SC_ - d, 1), pltpu.roll(v, d, 1))
    bk = (row & k) if k < SR_ else (lane & (k // SR_))
    keep_min = jnp.logical_not(jnp.logical_xor(bs == 0, bk == 0))
    le = x <= xp
    nlt = jnp.logical_not(x < xp)
    keep_self = jnp.logical_or(jnp.logical_and(keep_min, le),
                               jnp.logical_and(jnp.logical_not(keep_min), nlt))
    return jnp.where(keep_self, x, xp), jnp.where(keep_self, v, vp)


def _sort_kernel(y_ref, p_ref, u_ref, r_ref):
    x = -y_ref[:, :]
    v = p_ref[:, :]
    k = 2
    while k <= SR_ * SC_:
        s = k // 2
        while s >= 1:
            x, v = _cmpex(x, v, s, k)
            s //= 2
        k *= 2
    u_ref[:, :] = -x.T
    r_ref[:, :] = v.T


def _row128(e):
    acc = e[:, 0:128]
    for c in range(1, e.shape[1] // 128):
        acc = acc + e[:, c * 128:(c + 1) * 128]
    return acc


def _col8(e):
    h = e.shape[0]
    while h > 8:
        h //= 2
        e = e[:h, :] + e[h:2 * h, :]
    return e


def _loss_kernel(n, nb, u_col, r_col, u_row, r_row, loss_ref, rowacc, colacc):
    ib = pl.program_id(0)

    @pl.when(ib == 0)
    def _init():
        rowacc[:, :] = jnp.zeros_like(rowacc)
        colacc[:, :] = jnp.zeros_like(colacc)

    rc = r_col[pl.ds(ib * BT_, BT_), :]
    mrc = MARGIN_ - rc

    rr_d = r_row[:, pl.ds(ib * BT_, BT_)]
    e_d = jnp.maximum(mrc + rr_d, 0.0)
    ri = lax.broadcasted_iota(jnp.int32, (BT_, BT_), 0)
    ci = lax.broadcasted_iota(jnp.int32, (BT_, BT_), 1)
    e_d = jnp.where(ci > ri, e_d, 0.0)
    rowacc[pl.ds(ib * BT_, BT_), :] += _row128(e_d)
    colacc[:, pl.ds(ib * BT_, BT_)] += _col8(e_d)

    def body(jb, _):
        rr = r_row[:, pl.ds(jb * BT_, BT_)]
        e = jnp.maximum(mrc + rr, 0.0)
        rowacc[pl.ds(ib * BT_, BT_), :] += _row128(e)
        colacc[:, pl.ds(jb * BT_, BT_)] += _col8(e)
        return 0

    lax.fori_loop(ib + 1, nb, body, 0)

    @pl.when(ib == nb - 1)
    def _final():
        num = jnp.sum(rowacc[:, :] * u_col[:, :]) - jnp.sum(
            colacc[:, :] * u_row[:, :])
        idx = lax.broadcasted_iota(jnp.int32, (1, n), 1)
        coef = ((n - 1) - 2 * idx).astype(jnp.float32)
        den = jnp.sum(u_row[:, :] * coef)
        loss_ref[0, 0] = num / (den + 1e-8)


@jax.jit
def kernel(pred, y_true):
    p = pred.reshape(-1).astype(jnp.float32)
    y = y_true.reshape(-1).astype(jnp.float32)
    n = p.shape[0]
    nb = n // BT_

    u2, r2 = pl.pallas_call(
        _sort_kernel,
        out_shape=[
            jax.ShapeDtypeStruct((SC_, SR_), jnp.float32),
            jax.ShapeDtypeStruct((SC_, SR_), jnp.float32),
        ],
    )(y.reshape(SR_, SC_), p.reshape(SR_, SC_))
    u = u2.reshape(-1)
    r = r2.reshape(-1)

    loss = pl.pallas_call(
        functools.partial(_loss_kernel, n, nb),
        grid=(nb,),
        in_specs=[
            pl.BlockSpec((n, 1), lambda i: (0, 0)),
            pl.BlockSpec((n, 1), lambda i: (0, 0)),
            pl.BlockSpec((1, n), lambda i: (0, 0)),
            pl.BlockSpec((1, n), lambda i: (0, 0)),
        ],
        out_specs=pl.BlockSpec(memory_space=pltpu.SMEM),
        out_shape=jax.ShapeDtypeStruct((1, 1), jnp.float32),
        scratch_shapes=[
            pltpu.VMEM((n, 128), jnp.float32),
            pltpu.VMEM((8, n), jnp.float32),
        ],
    )(u.reshape(n, 1), r.reshape(n, 1), u.reshape(1, n), r.reshape(1, n))

    return loss[0, 0]


# speedup vs baseline: 6.9967x; 1.2790x over previous
import functools

import jax
import jax.numpy as jnp
from jax import lax
from jax.experimental import pallas as pl
from jax.experimental.pallas import tpu as pltpu

MARGIN_ = 0.02
BT_ = 512


def _row128(e):
    acc = e[:, 0:128]
    for c in range(1, e.shape[1] // 128):
        acc = acc + e[:, c * 128:(c + 1) * 128]
    return acc.astype(jnp.float32)


def _col8(e):
    h = e.shape[0]
    while h > 16:
        h //= 2
        e = e[:h, :] + e[h:2 * h, :]
    e = e.astype(jnp.float32)
    return e[:8, :] + e[8:16, :]


def _loss_kernel(n, nb, u_col, r_col, u_row, r_row, loss_ref, rowacc, colacc):
    ib = pl.program_id(0)

    @pl.when(ib == 0)
    def _init():
        rowacc[:, :] = jnp.zeros_like(rowacc)
        colacc[:, :] = jnp.zeros_like(colacc)

    rc = r_col[pl.ds(ib * BT_, BT_), :]
    mrc = (MARGIN_ - rc).astype(jnp.bfloat16)
    zero = jnp.bfloat16(0.0)

    rr_d = r_row[:, pl.ds(ib * BT_, BT_)].astype(jnp.bfloat16)
    e_d = jnp.maximum(mrc + rr_d, zero)
    ri = lax.broadcasted_iota(jnp.int32, (BT_, BT_), 0)
    ci = lax.broadcasted_iota(jnp.int32, (BT_, BT_), 1)
    e_d = jnp.where(ci > ri, e_d, zero)
    rowacc[pl.ds(ib * BT_, BT_), :] += _row128(e_d)
    colacc[:, pl.ds(ib * BT_, BT_)] += _col8(e_d)

    def body(jb, _):
        rr = r_row[:, pl.ds(jb * BT_, BT_)].astype(jnp.bfloat16)
        e = jnp.maximum(mrc + rr, zero)
        rowacc[pl.ds(ib * BT_, BT_), :] += _row128(e)
        colacc[:, pl.ds(jb * BT_, BT_)] += _col8(e)
        return 0

    lax.fori_loop(ib + 1, nb, body, 0)

    @pl.when(ib == nb - 1)
    def _final():
        num = jnp.sum(rowacc[:, :] * u_col[:, :]) - jnp.sum(
            colacc[:, :] * u_row[:, :])
        idx = lax.broadcasted_iota(jnp.int32, (1, n), 1)
        coef = ((n - 1) - 2 * idx).astype(jnp.float32)
        den = jnp.sum(u_row[:, :] * coef)
        loss_ref[0, 0] = num / (den + 1e-8)


@jax.jit
def kernel(pred, y_true):
    p = pred.reshape(-1).astype(jnp.float32)
    y = y_true.reshape(-1).astype(jnp.float32)
    n = p.shape[0]
    nb = n // BT_

    neg_u, r = lax.sort((-y, p), num_keys=1)
    u = -neg_u

    loss = pl.pallas_call(
        functools.partial(_loss_kernel, n, nb),
        grid=(nb,),
        in_specs=[
            pl.BlockSpec((n, 1), lambda i: (0, 0)),
            pl.BlockSpec((n, 1), lambda i: (0, 0)),
            pl.BlockSpec((1, n), lambda i: (0, 0)),
            pl.BlockSpec((1, n), lambda i: (0, 0)),
        ],
        out_specs=pl.BlockSpec(memory_space=pltpu.SMEM),
        out_shape=jax.ShapeDtypeStruct((1, 1), jnp.float32),
        scratch_shapes=[
            pltpu.VMEM((n, 128), jnp.float32),
            pltpu.VMEM((8, n), jnp.float32),
        ],
    )(u.reshape(n, 1), r.reshape(n, 1), u.reshape(1, n), r.reshape(1, n))

    return loss[0, 0]
